# baseline (speedup 1.0000x reference)
import jax
import jax.numpy as jnp
from jax import lax
from jax.experimental import pallas as pl
from jax.experimental.pallas import tpu as pltpu
from jax.experimental.pallas import tpu_sc as plsc

D_MODEL = 64
SCALE = 8.0
CH = 128
NBUF = 4
LANES = 16
NC, NS = 2, 16
NW = NC * NS


def kernel(x, table):
    seq, tok = x.shape
    n = seq * tok
    st_blocks = seq // CH
    idx = jnp.transpose(x).reshape(n)
    n_per_w = n // NW
    n_ch = n_per_w // CH
    mesh = plsc.VectorSubcoreMesh(core_axis_name="c", subcore_axis_name="s")

    @pl.kernel(
        out_type=jax.ShapeDtypeStruct(
            (tok, D_MODEL // 8, st_blocks, 8, CH), jnp.float32),
        mesh=mesh,
        scratch_types=[
            pltpu.VMEM((n_per_w,), jnp.int32),
            pltpu.VMEM((NBUF, CH, D_MODEL), jnp.float32),
            pltpu.VMEM((NBUF, D_MODEL // 8, 8, CH), jnp.float32),
            pltpu.SemaphoreType.DMA((NBUF,)),
            pltpu.SemaphoreType.DMA((NBUF,)),
        ],
        compiler_params=pltpu.CompilerParams(
            use_tc_tiling_on_sc=False, needs_layout_passes=False),
    )
    def gather_kernel(table_hbm, idx_hbm, out_hbm, idx_v, rows_g, trans,
                      gsem, osem):
        wid = lax.axis_index("s") * NC + lax.axis_index("c")
        base_c = wid * n_ch
        pltpu.sync_copy(idx_hbm.at[pl.ds(wid * n_per_w, n_per_w)], idx_v)
        iota16 = lax.iota(jnp.int32, LANES)

        def start_gather(k, b):
            pltpu.make_async_copy(
                table_hbm.at[idx_v.at[pl.ds(k * CH, CH)]],
                rows_g.at[b], gsem.at[b]).start()

        def wait_gather(b):
            pltpu.make_async_copy(
                table_hbm.at[idx_v.at[pl.ds(0, CH)]],
                rows_g.at[b], gsem.at[b]).wait()

        def start_out(k, b):
            gc = base_c + k
            t = gc // st_blocks
            st = gc % st_blocks
            pltpu.make_async_copy(
                trans.at[b], out_hbm.at[t, :, st], osem.at[b]).start()

        def wait_out(b):
            pltpu.make_async_copy(
                trans.at[b], out_hbm.at[0, :, 0], osem.at[b]).wait()

        siv_list = [iota16 + sc * LANES for sc in range(CH // LANES)]

        def transpose_scale(b):
            src = rows_g.at[b]

            @pl.loop(0, D_MODEL)
            def _(f):
                fv = jnp.full((LANES,), f, jnp.int32)
                ft = f // 8
                fi = f % 8
                for sc in range(CH // LANES):
                    v = plsc.load_gather(src, [siv_list[sc], fv])
                    trans.at[b, ft, fi,
                             pl.ds(sc * LANES, LANES)][...] = v * SCALE

        for b in range(NBUF):
            start_gather(b, b)
        for b in range(NBUF):
            wait_gather(b)
            transpose_scale(b)
            start_out(b, b)
            start_gather(b + NBUF, b)

        @pl.loop(NBUF, n_ch - NBUF, step=NBUF)
        def _(g):
            for b in range(NBUF):
                wait_gather(b)
                wait_out(b)
                transpose_scale(b)
                start_out(g + b, b)
                start_gather(g + b + NBUF, b)

        for b in range(NBUF):
            k = n_ch - NBUF + b
            wait_gather(b)
            wait_out(b)
            transpose_scale(b)
            start_out(k, b)
        for b in range(NBUF):
            wait_out(b)

    out5 = gather_kernel(table, idx)
    return out5.transpose((2, 4, 0, 1, 3)).reshape(seq, tok, D_MODEL)


# speedup vs baseline: 1.1248x; 1.1248x over previous
import jax
import jax.numpy as jnp
from jax import lax
from jax.experimental import pallas as pl
from jax.experimental.pallas import tpu as pltpu
from jax.experimental.pallas import tpu_sc as plsc

D_MODEL = 64
SCALE = 8.0
CH = 128
NBUF = 4
LANES = 16
NC, NS = 2, 16
NW = NC * NS


def kernel(x, table):
    seq, tok = x.shape
    n = seq * tok
    st_blocks = seq // CH
    idx = jnp.transpose(x).reshape(n)
    n_per_w = n // NW
    n_ch = n_per_w // CH
    mesh = plsc.VectorSubcoreMesh(core_axis_name="c", subcore_axis_name="s")

    @pl.kernel(
        out_type=jax.ShapeDtypeStruct(
            (tok, D_MODEL // 8, st_blocks, 8, CH), jnp.float32),
        mesh=mesh,
        scratch_types=[
            pltpu.VMEM((n_per_w,), jnp.int32),
            pltpu.VMEM((NBUF, CH, D_MODEL), jnp.float32),
            pltpu.VMEM((NBUF, D_MODEL, CH), jnp.float32),
            pltpu.SemaphoreType.DMA((NBUF,)),
            pltpu.SemaphoreType.DMA((NBUF,)),
        ],
        compiler_params=pltpu.CompilerParams(
            use_tc_tiling_on_sc=False, needs_layout_passes=False),
    )
    def gather_kernel(table_hbm, idx_hbm, out_hbm, idx_v, rows_g, trans,
                      gsem, osem):
        wid = lax.axis_index("s") * NC + lax.axis_index("c")
        base_c = wid * n_ch
        pltpu.sync_copy(idx_hbm.at[pl.ds(wid * n_per_w, n_per_w)], idx_v)
        iota16 = lax.iota(jnp.int32, LANES)
        fv_list = [iota16 + f0 for f0 in range(0, D_MODEL, LANES)]

        def start_gather(k, b):
            pltpu.make_async_copy(
                table_hbm.at[idx_v.at[pl.ds(k * CH, CH)]],
                rows_g.at[b], gsem.at[b]).start()

        def wait_gather(b):
            pltpu.make_async_copy(
                table_hbm.at[idx_v.at[pl.ds(0, CH)]],
                rows_g.at[b], gsem.at[b]).wait()

        def start_out(k, b):
            gc = base_c + k
            t = gc // st_blocks
            st = gc % st_blocks
            for ft in range(D_MODEL // 8):
                pltpu.make_async_copy(
                    trans.at[b, pl.ds(ft * 8, 8), :],
                    out_hbm.at[t, ft, st], osem.at[b]).start()

        def wait_out(b):
            for ft in range(D_MODEL // 8):
                pltpu.make_async_copy(
                    trans.at[b, pl.ds(0, 8), :],
                    out_hbm.at[0, 0, 0], osem.at[b]).wait()

        def transpose_scale(b):
            src = rows_g.at[b]
            dst = trans.at[b]

            @pl.loop(0, CH, step=4)
            def _(si):
                for u in range(4):
                    s = si + u
                    sb = lax.broadcast_in_dim(s, (LANES,), ())
                    for k in range(D_MODEL // LANES):
                        v = src.at[s, pl.ds(k * LANES, LANES)][...]
                        plsc.store_scatter(dst, [fv_list[k], sb], v * SCALE)

        for b in range(NBUF):
            start_gather(b, b)
        for b in range(NBUF):
            wait_gather(b)
            transpose_scale(b)
            start_out(b, b)
            start_gather(b + NBUF, b)

        @pl.loop(NBUF, n_ch - NBUF, step=NBUF)
        def _(g):
            for b in range(NBUF):
                wait_gather(b)
                wait_out(b)
                transpose_scale(b)
                start_out(g + b, b)
                start_gather(g + b + NBUF, b)

        for b in range(NBUF):
            k = n_ch - NBUF + b
            wait_gather(b)
            wait_out(b)
            transpose_scale(b)
            start_out(k, b)
        for b in range(NBUF):
            wait_out(b)

    out5 = gather_kernel(table, idx)
    return out5.transpose((2, 4, 0, 1, 3)).reshape(seq, tok, D_MODEL)


# speedup vs baseline: 1.4133x; 1.2565x over previous
import jax
import jax.numpy as jnp
from jax import lax
from jax.experimental import pallas as pl
from jax.experimental.pallas import tpu as pltpu
from jax.experimental.pallas import tpu_sc as plsc

D_MODEL = 64
SCALE = 8.0
CH = 128
NBUF = 4
LANES = 16
NC, NS = 2, 16
NW = NC * NS


def kernel(x, table):
    seq, tok = x.shape
    n = seq * tok
    st_blocks = seq // CH
    idx = jnp.transpose(x).reshape(n)
    n_per_w = n // NW
    n_ch = n_per_w // CH
    mesh = plsc.VectorSubcoreMesh(core_axis_name="c", subcore_axis_name="s")

    @pl.kernel(
        out_type=jax.ShapeDtypeStruct(
            (tok, D_MODEL // 8, st_blocks, 8, CH), jnp.float32),
        mesh=mesh,
        scratch_types=[
            pltpu.VMEM((n_per_w,), jnp.int32),
            pltpu.VMEM((NBUF, CH, D_MODEL), jnp.float32),
            pltpu.VMEM((NBUF, D_MODEL, CH), jnp.float32),
            pltpu.SemaphoreType.DMA((NBUF,)),
            pltpu.SemaphoreType.DMA((NBUF,)),
        ],
        compiler_params=pltpu.CompilerParams(
            use_tc_tiling_on_sc=False, needs_layout_passes=False),
    )
    def gather_kernel(table_hbm, idx_hbm, out_hbm, idx_v, rows_g, trans,
                      gsem, osem):
        wid = lax.axis_index("s") * NC + lax.axis_index("c")
        base_c = wid * n_ch
        pltpu.sync_copy(idx_hbm.at[pl.ds(wid * n_per_w, n_per_w)], idx_v)
        iota16 = lax.iota(jnp.int32, LANES)
        fv_list = [iota16 + f0 for f0 in range(0, D_MODEL, LANES)]

        def start_gather(k, b):
            pltpu.make_async_copy(
                table_hbm.at[idx_v.at[pl.ds(k * CH, CH)]],
                rows_g.at[b], gsem.at[b]).start()

        def wait_gather(b):
            pltpu.make_async_copy(
                table_hbm.at[idx_v.at[pl.ds(0, CH)]],
                rows_g.at[b], gsem.at[b]).wait()

        def start_out(k, b):
            gc = base_c + k
            t = gc // st_blocks
            st = gc % st_blocks
            for ft in range(D_MODEL // 8):
                pltpu.make_async_copy(
                    trans.at[b, pl.ds(ft * 8, 8), :],
                    out_hbm.at[t, ft, st], osem.at[b]).start()

        def wait_out(b):
            for ft in range(D_MODEL // 8):
                pltpu.make_async_copy(
                    trans.at[b, pl.ds(0, 8), :],
                    out_hbm.at[0, 0, 0], osem.at[b]).wait()

        def transpose_scale(b):
            src = rows_g.at[b]
            dst = trans.at[b]

            @plsc.parallel_loop(0, CH, step=4, unroll=2)
            def _(si):
                for u in range(4):
                    s = si + u
                    sb = lax.broadcast_in_dim(s, (LANES,), ())
                    for k in range(D_MODEL // LANES):
                        v = src.at[s, pl.ds(k * LANES, LANES)][...]
                        plsc.store_scatter(dst, [fv_list[k], sb], v * SCALE)

        for b in range(NBUF):
            start_gather(b, b)
        for b in range(NBUF):
            wait_gather(b)
            transpose_scale(b)
            start_out(b, b)
            start_gather(b + NBUF, b)

        @pl.loop(NBUF, n_ch - NBUF, step=NBUF)
        def _(g):
            for b in range(NBUF):
                wait_gather(b)
                wait_out(b)
                transpose_scale(b)
                start_out(g + b, b)
                start_gather(g + b + NBUF, b)

        for b in range(NBUF):
            k = n_ch - NBUF + b
            wait_gather(b)
            wait_out(b)
            transpose_scale(b)
            start_out(k, b)
        for b in range(NBUF):
            wait_out(b)

    out5 = gather_kernel(table, idx)
    return out5.transpose((2, 4, 0, 1, 3)).reshape(seq, tok, D_MODEL)


# speedup vs baseline: 1.4658x; 1.0372x over previous
import jax
import jax.numpy as jnp
from jax import lax
from jax.experimental import pallas as pl
from jax.experimental.pallas import tpu as pltpu
from jax.experimental.pallas import tpu_sc as plsc

D_MODEL = 64
SCALE = 8.0
CH = 128
NBUF = 4
LANES = 16
NC, NS = 2, 16
NW = NC * NS


def kernel(x, table):
    seq, tok = x.shape
    n = seq * tok
    st_blocks = seq // CH
    idx = jnp.transpose(x).reshape(n)
    n_per_w = n // NW
    n_ch = n_per_w // CH
    mesh = plsc.VectorSubcoreMesh(core_axis_name="c", subcore_axis_name="s")

    @pl.kernel(
        out_type=jax.ShapeDtypeStruct(
            (tok, D_MODEL // 8, st_blocks, 8, CH), jnp.float32),
        mesh=mesh,
        scratch_types=[
            pltpu.VMEM((n_per_w,), jnp.int32),
            pltpu.VMEM((NBUF, CH, D_MODEL), jnp.float32),
            pltpu.VMEM((NBUF, D_MODEL, CH), jnp.float32),
            pltpu.SemaphoreType.DMA((NBUF,)),
            pltpu.SemaphoreType.DMA((NBUF,)),
        ],
        compiler_params=pltpu.CompilerParams(
            use_tc_tiling_on_sc=False, needs_layout_passes=False),
    )
    def gather_kernel(table_hbm, idx_hbm, out_hbm, idx_v, rows_g, trans,
                      gsem, osem):
        wid = lax.axis_index("s") * NC + lax.axis_index("c")
        base_c = wid * n_ch
        pltpu.sync_copy(idx_hbm.at[pl.ds(wid * n_per_w, n_per_w)], idx_v)
        iota16 = lax.iota(jnp.int32, LANES)
        fv_list = [iota16 + f0 for f0 in range(0, D_MODEL, LANES)]

        def start_gather(k, b):
            pltpu.make_async_copy(
                table_hbm.at[idx_v.at[pl.ds(k * CH, CH)]],
                rows_g.at[b], gsem.at[b]).start()

        def wait_gather(b):
            pltpu.make_async_copy(
                table_hbm.at[idx_v.at[pl.ds(0, CH)]],
                rows_g.at[b], gsem.at[b]).wait()

        def start_out(k, b):
            gc = base_c + k
            t = gc // st_blocks
            st = gc % st_blocks
            for ft in range(D_MODEL // 8):
                pltpu.make_async_copy(
                    trans.at[b, pl.ds(ft * 8, 8), :],
                    out_hbm.at[t, ft, st], osem.at[b]).start()

        def wait_out(b):
            for ft in range(D_MODEL // 8):
                pltpu.make_async_copy(
                    trans.at[b, pl.ds(0, 8), :],
                    out_hbm.at[0, 0, 0], osem.at[b]).wait()

        def transpose_scale(b):
            src = rows_g.at[b]
            dst = trans.at[b]

            @plsc.parallel_loop(0, CH, step=4, unroll=1)
            def _(si):
                for u in range(4):
                    s = si + u
                    sb = lax.broadcast_in_dim(s, (LANES,), ())
                    for k in range(D_MODEL // LANES):
                        v = src.at[s, pl.ds(k * LANES, LANES)][...]
                        plsc.store_scatter(dst, [fv_list[k], sb], v * SCALE)

        for b in range(NBUF):
            start_gather(b, b)

        @pl.loop(0, n_ch)
        def _(k):
            b = lax.rem(k, NBUF)
            wait_gather(b)

            @pl.when(k >= NBUF)
            def _():
                wait_out(b)

            transpose_scale(b)
            start_out(k, b)

            @pl.when(k + NBUF < n_ch)
            def _():
                start_gather(k + NBUF, b)

        for b in range(NBUF):
            wait_out(b)

    out5 = gather_kernel(table, idx)
    return out5.transpose((2, 4, 0, 1, 3)).reshape(seq, tok, D_MODEL)


# speedup vs baseline: 1.4691x; 1.0022x over previous
import jax
import jax.numpy as jnp
from jax import lax
from jax.experimental import pallas as pl
from jax.experimental.pallas import tpu as pltpu
from jax.experimental.pallas import tpu_sc as plsc

D_MODEL = 64
SCALE = 8.0
CH = 128
NBUF = 4
LANES = 16
NC, NS = 2, 16
NW = NC * NS


def kernel(x, table):
    seq, tok = x.shape
    n = seq * tok
    st_blocks = seq // CH
    idx = jnp.transpose(x).reshape(n)
    n_per_w = n // NW
    n_ch = n_per_w // CH
    mesh = plsc.VectorSubcoreMesh(core_axis_name="c", subcore_axis_name="s")

    @pl.kernel(
        out_type=jax.ShapeDtypeStruct(
            (tok, D_MODEL // 8, st_blocks, 8, CH), jnp.float32),
        mesh=mesh,
        scratch_types=[
            pltpu.VMEM((n_per_w,), jnp.int32),
            pltpu.VMEM((NBUF, CH, D_MODEL), jnp.float32),
            pltpu.VMEM((NBUF, D_MODEL, CH), jnp.float32),
            pltpu.SemaphoreType.DMA((NBUF,)),
            pltpu.SemaphoreType.DMA((NBUF,)),
        ],
        compiler_params=pltpu.CompilerParams(
            use_tc_tiling_on_sc=False, needs_layout_passes=False),
    )
    def gather_kernel(table_hbm, idx_hbm, out_hbm, idx_v, rows_g, trans,
                      gsem, osem):
        wid = lax.axis_index("s") * NC + lax.axis_index("c")
        base_c = wid * n_ch
        pltpu.sync_copy(idx_hbm.at[pl.ds(wid * n_per_w, n_per_w)], idx_v)
        iota16 = lax.iota(jnp.int32, LANES)
        fv_list = [iota16 + f0 for f0 in range(0, D_MODEL, LANES)]

        def start_gather(k, b):
            pltpu.make_async_copy(
                table_hbm.at[idx_v.at[pl.ds(k * CH, CH)]],
                rows_g.at[b], gsem.at[b]).start()

        def wait_gather(b):
            pltpu.make_async_copy(
                table_hbm.at[idx_v.at[pl.ds(0, CH)]],
                rows_g.at[b], gsem.at[b]).wait()

        def start_out(k, b):
            gc = base_c + k
            t = gc // st_blocks
            st = gc % st_blocks
            for ft in range(D_MODEL // 8):
                pltpu.make_async_copy(
                    trans.at[b, pl.ds(ft * 8, 8), :],
                    out_hbm.at[t, ft, st], osem.at[b]).start()

        def wait_out(b):
            for ft in range(D_MODEL // 8):
                pltpu.make_async_copy(
                    trans.at[b, pl.ds(0, 8), :],
                    out_hbm.at[0, 0, 0], osem.at[b]).wait()

        def transpose_scale(b):
            src = rows_g.at[b]
            dst = trans.at[b]

            @plsc.parallel_loop(0, CH, step=1, unroll=8)
            def _(s):
                sb = lax.broadcast_in_dim(s, (LANES,), ())
                for k in range(D_MODEL // LANES):
                    v = src.at[s, pl.ds(k * LANES, LANES)][...]
                    plsc.store_scatter(dst, [fv_list[k], sb], v * SCALE)

        for b in range(NBUF):
            start_gather(b, b)

        @pl.loop(0, n_ch)
        def _(k):
            b = lax.rem(k, NBUF)
            wait_gather(b)

            @pl.when(k >= NBUF)
            def _():
                wait_out(b)

            transpose_scale(b)
            start_out(k, b)

            @pl.when(k + NBUF < n_ch)
            def _():
                start_gather(k + NBUF, b)

        for b in range(NBUF):
            wait_out(b)

    out5 = gather_kernel(table, idx)
    return out5.transpose((2, 4, 0, 1, 3)).reshape(seq, tok, D_MODEL)


# speedup vs baseline: 2.5806x; 1.7566x over previous
import jax
import jax.numpy as jnp
from jax import lax
from jax.experimental import pallas as pl
from jax.experimental.pallas import tpu as pltpu
from jax.experimental.pallas import tpu_sc as plsc

D_MODEL = 64
SCALE = 8.0
CH = 128
NBUF = 4
LANES = 16
NC, NS = 2, 16
NW = NC * NS


def kernel(x, table):
    seq, tok = x.shape
    n = seq * tok
    st_blocks = seq // CH
    idx = jnp.transpose(x).reshape(n)
    n_per_w = n // NW
    n_ch = n_per_w // CH
    mesh = plsc.VectorSubcoreMesh(core_axis_name="c", subcore_axis_name="s")

    @pl.kernel(
        out_type=jax.ShapeDtypeStruct(
            (tok, D_MODEL // 8, st_blocks, 8, CH), jnp.float32),
        mesh=mesh,
        scratch_types=[
            pltpu.VMEM((n_per_w,), jnp.int32),
            pltpu.VMEM((NBUF, CH, D_MODEL), jnp.float32),
            pltpu.VMEM((NBUF, D_MODEL, CH + 1), jnp.float32),
            pltpu.SemaphoreType.DMA((NBUF,)),
            pltpu.SemaphoreType.DMA((NBUF,)),
        ],
        compiler_params=pltpu.CompilerParams(
            use_tc_tiling_on_sc=False, needs_layout_passes=False),
    )
    def gather_kernel(table_hbm, idx_hbm, out_hbm, idx_v, rows_g, trans,
                      gsem, osem):
        wid = lax.axis_index("s") * NC + lax.axis_index("c")
        base_c = wid * n_ch
        pltpu.sync_copy(idx_hbm.at[pl.ds(wid * n_per_w, n_per_w)], idx_v)
        iota16 = lax.iota(jnp.int32, LANES)
        fv_list = [iota16 + f0 for f0 in range(0, D_MODEL, LANES)]

        def start_gather(k, b):
            pltpu.make_async_copy(
                table_hbm.at[idx_v.at[pl.ds(k * CH, CH)]],
                rows_g.at[b], gsem.at[b]).start()

        def wait_gather(b):
            pltpu.make_async_copy(
                table_hbm.at[idx_v.at[pl.ds(0, CH)]],
                rows_g.at[b], gsem.at[b]).wait()

        def start_out(k, b):
            gc = base_c + k
            t = gc // st_blocks
            st = gc % st_blocks
            for ft in range(D_MODEL // 8):
                pltpu.make_async_copy(
                    trans.at[b, pl.ds(ft * 8, 8), pl.ds(0, CH)],
                    out_hbm.at[t, ft, st], osem.at[b]).start()

        def wait_out(b):
            for ft in range(D_MODEL // 8):
                pltpu.make_async_copy(
                    trans.at[b, pl.ds(0, 8), pl.ds(0, CH)],
                    out_hbm.at[0, 0, 0], osem.at[b]).wait()

        def transpose_scale(b):
            src = rows_g.at[b]
            dst = trans.at[b]

            @plsc.parallel_loop(0, CH, step=1, unroll=8)
            def _(s):
                sb = lax.broadcast_in_dim(s, (LANES,), ())
                for k in range(D_MODEL // LANES):
                    v = src.at[s, pl.ds(k * LANES, LANES)][...]
                    plsc.store_scatter(dst, [fv_list[k], sb], v * SCALE)

        for b in range(NBUF):
            start_gather(b, b)

        @pl.loop(0, n_ch)
        def _(k):
            b = lax.rem(k, NBUF)
            wait_gather(b)

            @pl.when(k >= NBUF)
            def _():
                wait_out(b)

            transpose_scale(b)
            start_out(k, b)

            @pl.when(k + NBUF < n_ch)
            def _():
                start_gather(k + NBUF, b)

        for b in range(NBUF):
            wait_out(b)

    out5 = gather_kernel(table, idx)
    return out5.transpose((2, 4, 0, 1, 3)).reshape(seq, tok, D_MODEL)
